# baseline (speedup 1.0000x reference)
import functools

import jax
import jax.numpy as jnp
from jax import lax
from jax.experimental import pallas as pl
from jax.experimental.pallas import tpu as pltpu
from jax.experimental.pallas import tpu_sc as plsc

N_ELEM = 85
ZP = 96
N_TAG = 3
HIDDEN = 384
N_NODE = 10000

NC = 2
NS = 16
NW = NC * NS
BIG = 320
SMALL = 304
BIG_W = 17
GCHUNK = 80

E_TILE = 16000
NUM_FILT_H = 64


def _table_body(emb_ref, phys_ref, pp_ref, pg_ref, per_ref, grp_ref, tagt_ref,
                wlin_ref, blin_ref, out_ref):
    w1 = wlin_ref[0:272, :]
    w2 = wlin_ref[272:304, :]
    w3 = wlin_ref[304:320, :]
    w4 = wlin_ref[320:352, :]
    w5 = wlin_ref[352:384, :]
    f32 = jnp.float32
    pw = jnp.dot(per_ref[...], w4, preferred_element_type=f32)
    gw = jnp.dot(grp_ref[...], w5, preferred_element_type=f32)
    oh_p = (pp_ref[...] == lax.broadcasted_iota(jnp.int32, (ZP, 8), 1)).astype(f32)
    oh_g = (pg_ref[...] == lax.broadcasted_iota(jnp.int32, (ZP, 19), 1)).astype(f32)
    u = (jnp.dot(emb_ref[...], w1, preferred_element_type=f32)
         + jnp.dot(phys_ref[...], w3, preferred_element_type=f32)
         + jnp.dot(oh_p, pw, preferred_element_type=f32)
         + jnp.dot(oh_g, gw, preferred_element_type=f32)
         + blin_ref[...])
    v = jnp.dot(tagt_ref[...], w2, preferred_element_type=f32)
    for t in range(N_TAG):
        x = u + v[t:t + 1, :]
        out_ref[t * ZP:(t + 1) * ZP, :] = x * jax.nn.sigmoid(x)


def _build_table(emb96, phys96, pp96, pg96, period_table, group_table,
                 tag_table, w_lin, b_lin2):
    return pl.pallas_call(
        _table_body,
        out_shape=jax.ShapeDtypeStruct((N_TAG * ZP, HIDDEN), jnp.float32),
    )(emb96, phys96, pp96, pg96, period_table, group_table, tag_table,
      w_lin, b_lin2)


def _sc_body(table_hbm, z_hbm, tag_hbm, out_hbm, z_v, tag_v, idx_v, rows_v, sem):
    wid = lax.axis_index("s") * NC + lax.axis_index("c")
    is_big = wid < BIG_W
    base = wid * SMALL + 16 * jnp.minimum(wid, BIG_W)

    @pl.when(is_big)
    def _():
        pltpu.sync_copy(z_hbm.at[pl.ds(base, BIG)], z_v)
        pltpu.sync_copy(tag_hbm.at[pl.ds(base, BIG)], tag_v)

    @pl.when(jnp.logical_not(is_big))
    def _():
        pltpu.sync_copy(z_hbm.at[pl.ds(base, SMALL)], z_v.at[pl.ds(0, SMALL)])
        pltpu.sync_copy(tag_hbm.at[pl.ds(base, SMALL)],
                        tag_v.at[pl.ds(0, SMALL)])

    for j in range(BIG // GCHUNK):
        for r in range(GCHUNK // 16):
            off = j * GCHUNK + r * 16
            zc = z_v[pl.ds(off, 16)]
            tc = tag_v[pl.ds(off, 16)]
            idx_v[j, pl.ds(r * 16, 16)] = tc * ZP + zc
    copies = []
    for j in range(3):
        copies.append(
            pltpu.async_copy(table_hbm.at[idx_v.at[j]],
                             rows_v.at[pl.ds(j * GCHUNK, GCHUNK)], sem))
    for c in copies:
        c.wait()

    @pl.when(is_big)
    def _():
        pltpu.async_copy(table_hbm.at[idx_v.at[3]],
                         rows_v.at[pl.ds(3 * GCHUNK, GCHUNK)], sem).wait()
        pltpu.sync_copy(rows_v, out_hbm.at[pl.ds(base, BIG)])

    @pl.when(jnp.logical_not(is_big))
    def _():
        pltpu.async_copy(table_hbm.at[idx_v.at[3, pl.ds(0, SMALL - 3 * GCHUNK)]],
                         rows_v.at[pl.ds(3 * GCHUNK, SMALL - 3 * GCHUNK)],
                         sem).wait()
        pltpu.sync_copy(rows_v.at[pl.ds(0, SMALL)],
                        out_hbm.at[pl.ds(base, SMALL)])


def _gather_nodes(table, z, tag):
    mesh = plsc.VectorSubcoreMesh(core_axis_name="c", subcore_axis_name="s")
    fn = pl.kernel(
        _sc_body,
        out_type=jax.ShapeDtypeStruct((N_NODE, HIDDEN), jnp.float32),
        mesh=mesh,
        scratch_types=[
            pltpu.VMEM((BIG,), jnp.int32),
            pltpu.VMEM((BIG,), jnp.int32),
            pltpu.VMEM((BIG // GCHUNK, GCHUNK), jnp.int32),
            pltpu.VMEM((BIG, HIDDEN), jnp.float32),
            pltpu.SemaphoreType.DMA,
        ],
    )
    return fn(table, z, tag)


def _edge_body(rpT_ref, eaT_ref, we1_ref, be1_ref, we12_ref, be12_ref,
               out_ref):
    dn = (((0,), (0,)), ((), ()))
    a = lax.dot_general(rpT_ref[...], we1_ref[...], dn,
                        preferred_element_type=jnp.float32) + be1_ref[...]
    b = lax.dot_general(eaT_ref[...], we12_ref[...], dn,
                        preferred_element_type=jnp.float32) + be12_ref[...]
    out_ref[:, 0:NUM_FILT_H] = a / (1.0 + jnp.exp(-a))
    out_ref[:, NUM_FILT_H:2 * NUM_FILT_H] = b / (1.0 + jnp.exp(-b))


def _edge_branch(rel_pos, edge_attr, w_e1, b_e1, w_e12, b_e12):
    e_rows = rel_pos.shape[0]
    grid = (e_rows // E_TILE,)
    ng = edge_attr.shape[1]
    rpT = rel_pos.T
    eaT = edge_attr.T
    return pl.pallas_call(
        _edge_body,
        grid=grid,
        in_specs=[
            pl.BlockSpec((3, E_TILE), lambda i: (0, i)),
            pl.BlockSpec((ng, E_TILE), lambda i: (0, i)),
            pl.BlockSpec((3, NUM_FILT_H), lambda i: (0, 0)),
            pl.BlockSpec((1, NUM_FILT_H), lambda i: (0, 0)),
            pl.BlockSpec((ng, NUM_FILT_H), lambda i: (0, 0)),
            pl.BlockSpec((1, NUM_FILT_H), lambda i: (0, 0)),
        ],
        out_specs=pl.BlockSpec((E_TILE, 2 * NUM_FILT_H), lambda i: (i, 0)),
        out_shape=jax.ShapeDtypeStruct((e_rows, 2 * NUM_FILT_H), jnp.float32),
    )(rpT, eaT, w_e1, b_e1.reshape(1, NUM_FILT_H),
      w_e12, b_e12.reshape(1, NUM_FILT_H))


def kernel(z, rel_pos, edge_attr, tag, emb_table, tag_table, period_table,
           group_table, phys_properties, phys_period, phys_group,
           W_lin, b_lin, W_e1, b_e1, W_e12, b_e12):
    pad_rows = ZP - N_ELEM
    emb96 = jnp.pad(emb_table, ((0, pad_rows), (0, 0)))
    phys96 = jnp.pad(phys_properties, ((0, pad_rows), (0, 0)))
    pp96 = jnp.pad(phys_period, (0, pad_rows)).reshape(ZP, 1)
    pg96 = jnp.pad(phys_group, (0, pad_rows)).reshape(ZP, 1)
    table = _build_table(emb96, phys96, pp96, pg96, period_table, group_table,
                         tag_table, W_lin, b_lin.reshape(1, HIDDEN))

    h = _gather_nodes(table, z, tag)

    e = _edge_branch(rel_pos, edge_attr, W_e1, b_e1, W_e12, b_e12)
    return (h, e)


# speedup vs baseline: 1.0049x; 1.0049x over previous
import functools

import jax
import jax.numpy as jnp
from jax import lax
from jax.experimental import pallas as pl
from jax.experimental.pallas import tpu as pltpu
from jax.experimental.pallas import tpu_sc as plsc

N_ELEM = 85
ZP = 96
N_TAG = 3
HIDDEN = 384
N_NODE = 10000

NC = 2
NS = 16
NW = NC * NS
BIG = 320
SMALL = 304
BIG_W = 17
GCHUNK = 80

E_TILE = 16000
NUM_FILT_H = 64


def _table_body(emb_ref, phys_ref, pp_ref, pg_ref, per_ref, grp_ref, tagt_ref,
                wlin_ref, blin_ref, out_ref):
    w1 = wlin_ref[0:272, :]
    w2 = wlin_ref[272:304, :]
    w3 = wlin_ref[304:320, :]
    w4 = wlin_ref[320:352, :]
    w5 = wlin_ref[352:384, :]
    f32 = jnp.float32
    pw = jnp.dot(per_ref[...], w4, preferred_element_type=f32)
    gw = jnp.dot(grp_ref[...], w5, preferred_element_type=f32)
    oh_p = (pp_ref[...] == lax.broadcasted_iota(jnp.int32, (N_ELEM, 8), 1)).astype(f32)
    oh_g = (pg_ref[...] == lax.broadcasted_iota(jnp.int32, (N_ELEM, 19), 1)).astype(f32)
    u = (jnp.dot(emb_ref[...], w1, preferred_element_type=f32)
         + jnp.dot(phys_ref[...], w3, preferred_element_type=f32)
         + jnp.dot(oh_p, pw, preferred_element_type=f32)
         + jnp.dot(oh_g, gw, preferred_element_type=f32)
         + blin_ref[...].reshape(1, HIDDEN))
    v = jnp.dot(tagt_ref[...], w2, preferred_element_type=f32)
    for t in range(N_TAG):
        x = u + v[t:t + 1, :]
        out_ref[t * ZP:t * ZP + N_ELEM, :] = x / (1.0 + jnp.exp(-x))


def _build_table(emb_table, phys_properties, pp_col, pg_col, period_table,
                 group_table, tag_table, w_lin, b_lin):
    return pl.pallas_call(
        _table_body,
        out_shape=jax.ShapeDtypeStruct((N_TAG * ZP, HIDDEN), jnp.float32),
    )(emb_table, phys_properties, pp_col, pg_col, period_table, group_table,
      tag_table, w_lin, b_lin)


def _sc_body(table_hbm, z_hbm, tag_hbm, out_hbm, z_v, tag_v, idx_v, rows_v, sem):
    wid = lax.axis_index("s") * NC + lax.axis_index("c")
    is_big = wid < BIG_W
    base = wid * SMALL + 16 * jnp.minimum(wid, BIG_W)

    @pl.when(is_big)
    def _():
        pltpu.sync_copy(z_hbm.at[pl.ds(base, BIG)], z_v)
        pltpu.sync_copy(tag_hbm.at[pl.ds(base, BIG)], tag_v)

    @pl.when(jnp.logical_not(is_big))
    def _():
        pltpu.sync_copy(z_hbm.at[pl.ds(base, SMALL)], z_v.at[pl.ds(0, SMALL)])
        pltpu.sync_copy(tag_hbm.at[pl.ds(base, SMALL)],
                        tag_v.at[pl.ds(0, SMALL)])

    for j in range(BIG // GCHUNK):
        for r in range(GCHUNK // 16):
            off = j * GCHUNK + r * 16
            zc = z_v[pl.ds(off, 16)]
            tc = tag_v[pl.ds(off, 16)]
            idx_v[j, pl.ds(r * 16, 16)] = tc * ZP + zc
    copies = []
    for j in range(3):
        copies.append(
            pltpu.async_copy(table_hbm.at[idx_v.at[j]],
                             rows_v.at[pl.ds(j * GCHUNK, GCHUNK)], sem))
    for c in copies:
        c.wait()

    @pl.when(is_big)
    def _():
        pltpu.async_copy(table_hbm.at[idx_v.at[3]],
                         rows_v.at[pl.ds(3 * GCHUNK, GCHUNK)], sem).wait()
        pltpu.sync_copy(rows_v, out_hbm.at[pl.ds(base, BIG)])

    @pl.when(jnp.logical_not(is_big))
    def _():
        pltpu.async_copy(table_hbm.at[idx_v.at[3, pl.ds(0, SMALL - 3 * GCHUNK)]],
                         rows_v.at[pl.ds(3 * GCHUNK, SMALL - 3 * GCHUNK)],
                         sem).wait()
        pltpu.sync_copy(rows_v.at[pl.ds(0, SMALL)],
                        out_hbm.at[pl.ds(base, SMALL)])


def _gather_nodes(table, z, tag):
    mesh = plsc.VectorSubcoreMesh(core_axis_name="c", subcore_axis_name="s")
    fn = pl.kernel(
        _sc_body,
        out_type=jax.ShapeDtypeStruct((N_NODE, HIDDEN), jnp.float32),
        mesh=mesh,
        scratch_types=[
            pltpu.VMEM((BIG,), jnp.int32),
            pltpu.VMEM((BIG,), jnp.int32),
            pltpu.VMEM((BIG // GCHUNK, GCHUNK), jnp.int32),
            pltpu.VMEM((BIG, HIDDEN), jnp.float32),
            pltpu.SemaphoreType.DMA,
        ],
    )
    return fn(table, z, tag)


def _edge_body(rpT_ref, eaT_ref, we1_ref, be1_ref, we12_ref, be12_ref,
               out_ref):
    dn = (((0,), (0,)), ((), ()))
    a = lax.dot_general(rpT_ref[...], we1_ref[...], dn,
                        preferred_element_type=jnp.float32) + be1_ref[...]
    b = lax.dot_general(eaT_ref[...], we12_ref[...], dn,
                        preferred_element_type=jnp.float32) + be12_ref[...]
    out_ref[:, 0:NUM_FILT_H] = a / (1.0 + jnp.exp(-a))
    out_ref[:, NUM_FILT_H:2 * NUM_FILT_H] = b / (1.0 + jnp.exp(-b))


def _edge_branch(rel_pos, edge_attr, w_e1, b_e1, w_e12, b_e12):
    e_rows = rel_pos.shape[0]
    grid = (e_rows // E_TILE,)
    ng = edge_attr.shape[1]
    rpT = rel_pos.T
    eaT = edge_attr.T
    return pl.pallas_call(
        _edge_body,
        grid=grid,
        in_specs=[
            pl.BlockSpec((3, E_TILE), lambda i: (0, i)),
            pl.BlockSpec((ng, E_TILE), lambda i: (0, i)),
            pl.BlockSpec((3, NUM_FILT_H), lambda i: (0, 0)),
            pl.BlockSpec((1, NUM_FILT_H), lambda i: (0, 0)),
            pl.BlockSpec((ng, NUM_FILT_H), lambda i: (0, 0)),
            pl.BlockSpec((1, NUM_FILT_H), lambda i: (0, 0)),
        ],
        out_specs=pl.BlockSpec((E_TILE, 2 * NUM_FILT_H), lambda i: (i, 0)),
        out_shape=jax.ShapeDtypeStruct((e_rows, 2 * NUM_FILT_H), jnp.float32),
    )(rpT, eaT, w_e1, b_e1.reshape(1, NUM_FILT_H),
      w_e12, b_e12.reshape(1, NUM_FILT_H))


def kernel(z, rel_pos, edge_attr, tag, emb_table, tag_table, period_table,
           group_table, phys_properties, phys_period, phys_group,
           W_lin, b_lin, W_e1, b_e1, W_e12, b_e12):
    table = _build_table(emb_table, phys_properties,
                         phys_period.reshape(N_ELEM, 1),
                         phys_group.reshape(N_ELEM, 1),
                         period_table, group_table,
                         tag_table, W_lin, b_lin)

    h = _gather_nodes(table, z, tag)

    e = _edge_branch(rel_pos, edge_attr, W_e1, b_e1, W_e12, b_e12)
    return (h, e)


# speedup vs baseline: 1.0091x; 1.0042x over previous
import jax
import jax.numpy as jnp
from jax import lax
from jax.experimental import pallas as pl
from jax.experimental.pallas import tpu as pltpu
from jax.experimental.pallas import tpu_sc as plsc

N_ELEM = 85
ZP = 96
N_TAG = 3
HIDDEN = 384
N_NODE = 10000

NC = 2
NS = 16
NW = NC * NS
BIG = 320
SMALL = 304
BIG_W = 17
GCHUNK = 80

E_TILE = 16000
NUM_FILT_H = 64


def _table_body(emb_ref, phys_ref, pp_ref, pg_ref, per_ref, grp_ref, tagt_ref,
                wlin_ref, blin_ref, out_ref):
    w1 = wlin_ref[0:272, :]
    w2 = wlin_ref[272:304, :]
    w3 = wlin_ref[304:320, :]
    w4 = wlin_ref[320:352, :]
    w5 = wlin_ref[352:384, :]
    f32 = jnp.float32
    pw = jnp.dot(per_ref[...], w4, preferred_element_type=f32)
    gw = jnp.dot(grp_ref[...], w5, preferred_element_type=f32)
    oh_p = (pp_ref[...] == lax.broadcasted_iota(jnp.int32, (N_ELEM, 8), 1)).astype(f32)
    oh_g = (pg_ref[...] == lax.broadcasted_iota(jnp.int32, (N_ELEM, 19), 1)).astype(f32)
    u = (jnp.dot(emb_ref[...], w1, preferred_element_type=f32)
         + jnp.dot(phys_ref[...], w3, preferred_element_type=f32)
         + jnp.dot(oh_p, pw, preferred_element_type=f32)
         + jnp.dot(oh_g, gw, preferred_element_type=f32)
         + blin_ref[...].reshape(1, HIDDEN))
    v = jnp.dot(tagt_ref[...], w2, preferred_element_type=f32)
    for t in range(N_TAG):
        x = u + v[t:t + 1, :]
        out_ref[t * ZP:t * ZP + N_ELEM, :] = x / (1.0 + jnp.exp(-x))


def _build_table(emb_table, phys_properties, pp_col, pg_col, period_table,
                 group_table, tag_table, w_lin, b_lin):
    return pl.pallas_call(
        _table_body,
        out_shape=jax.ShapeDtypeStruct((N_TAG * ZP, HIDDEN), jnp.float32),
    )(emb_table, phys_properties, pp_col, pg_col, period_table, group_table,
      tag_table, w_lin, b_lin)


def _sc_body(table_hbm, z_hbm, tag_hbm, out_hbm, z_v, tag_v, idx_v, rows_v, sem):
    wid = lax.axis_index("s") * NC + lax.axis_index("c")
    is_big = wid < BIG_W
    base = wid * SMALL + 16 * jnp.minimum(wid, BIG_W)

    @pl.when(is_big)
    def _():
        pltpu.sync_copy(z_hbm.at[pl.ds(base, BIG)], z_v)
        pltpu.sync_copy(tag_hbm.at[pl.ds(base, BIG)], tag_v)

    @pl.when(jnp.logical_not(is_big))
    def _():
        pltpu.sync_copy(z_hbm.at[pl.ds(base, SMALL)], z_v.at[pl.ds(0, SMALL)])
        pltpu.sync_copy(tag_hbm.at[pl.ds(base, SMALL)],
                        tag_v.at[pl.ds(0, SMALL)])

    for j in range(BIG // GCHUNK):
        for r in range(GCHUNK // 16):
            off = j * GCHUNK + r * 16
            zc = z_v[pl.ds(off, 16)]
            tc = tag_v[pl.ds(off, 16)]
            idx_v[j, pl.ds(r * 16, 16)] = tc * ZP + zc
    copies = []
    for j in range(3):
        copies.append(
            pltpu.async_copy(table_hbm.at[idx_v.at[j]],
                             rows_v.at[pl.ds(j * GCHUNK, GCHUNK)], sem))
    for c in copies:
        c.wait()

    @pl.when(is_big)
    def _():
        pltpu.async_copy(table_hbm.at[idx_v.at[3]],
                         rows_v.at[pl.ds(3 * GCHUNK, GCHUNK)], sem).wait()
        pltpu.sync_copy(rows_v, out_hbm.at[pl.ds(base, BIG)])

    @pl.when(jnp.logical_not(is_big))
    def _():
        pltpu.async_copy(table_hbm.at[idx_v.at[3, pl.ds(0, SMALL - 3 * GCHUNK)]],
                         rows_v.at[pl.ds(3 * GCHUNK, SMALL - 3 * GCHUNK)],
                         sem).wait()
        pltpu.sync_copy(rows_v.at[pl.ds(0, SMALL)],
                        out_hbm.at[pl.ds(base, SMALL)])


def _gather_nodes(table, z, tag):
    mesh = plsc.VectorSubcoreMesh(core_axis_name="c", subcore_axis_name="s")
    fn = pl.kernel(
        _sc_body,
        out_type=jax.ShapeDtypeStruct((N_NODE, HIDDEN), jnp.float32),
        mesh=mesh,
        scratch_types=[
            pltpu.VMEM((BIG,), jnp.int32),
            pltpu.VMEM((BIG,), jnp.int32),
            pltpu.VMEM((BIG // GCHUNK, GCHUNK), jnp.int32),
            pltpu.VMEM((BIG, HIDDEN), jnp.float32),
            pltpu.SemaphoreType.DMA,
        ],
    )
    return fn(table, z, tag)


def _edge_body(rpT_ref, eaT_ref, we1_ref, be1_ref, we12_ref, be12_ref,
               out_ref):
    dn = (((0,), (0,)), ((), ()))
    a = lax.dot_general(rpT_ref[...], we1_ref[...], dn,
                        preferred_element_type=jnp.float32) + be1_ref[...]
    b = lax.dot_general(eaT_ref[...], we12_ref[...], dn,
                        preferred_element_type=jnp.float32) + be12_ref[...]
    out_ref[:, 0:NUM_FILT_H] = a / (1.0 + jnp.exp(-a))
    out_ref[:, NUM_FILT_H:2 * NUM_FILT_H] = b / (1.0 + jnp.exp(-b))


def _edge_branch(rel_pos, edge_attr, w_e1, b_e1, w_e12, b_e12):
    e_rows = rel_pos.shape[0]
    grid = (e_rows // E_TILE,)
    ng = edge_attr.shape[1]
    rpT = rel_pos.T
    eaT = edge_attr.T
    return pl.pallas_call(
        _edge_body,
        grid=grid,
        in_specs=[
            pl.BlockSpec((3, E_TILE), lambda i: (0, i)),
            pl.BlockSpec((ng, E_TILE), lambda i: (0, i)),
            pl.BlockSpec((3, NUM_FILT_H), lambda i: (0, 0)),
            pl.BlockSpec((1, NUM_FILT_H), lambda i: (0, 0)),
            pl.BlockSpec((ng, NUM_FILT_H), lambda i: (0, 0)),
            pl.BlockSpec((1, NUM_FILT_H), lambda i: (0, 0)),
        ],
        out_specs=pl.BlockSpec((E_TILE, 2 * NUM_FILT_H), lambda i: (i, 0)),
        out_shape=jax.ShapeDtypeStruct((e_rows, 2 * NUM_FILT_H), jnp.float32),
    )(rpT, eaT, w_e1, b_e1.reshape(1, NUM_FILT_H),
      w_e12, b_e12.reshape(1, NUM_FILT_H))


def kernel(z, rel_pos, edge_attr, tag, emb_table, tag_table, period_table,
           group_table, phys_properties, phys_period, phys_group,
           W_lin, b_lin, W_e1, b_e1, W_e12, b_e12):
    table = _build_table(emb_table, phys_properties,
                         phys_period.reshape(N_ELEM, 1),
                         phys_group.reshape(N_ELEM, 1),
                         period_table, group_table,
                         tag_table, W_lin, b_lin)

    h = _gather_nodes(table, z, tag)

    e = _edge_branch(rel_pos, edge_attr, W_e1, b_e1, W_e12, b_e12)
    return (h, e)
